# baseline (speedup 1.0000x reference)
import functools

import jax
import jax.numpy as jnp
from jax import lax
from jax.experimental import pallas as pl
from jax.experimental.pallas import tpu as pltpu
from jax.experimental.pallas import tpu_sc as plsc

ROWS = 4096 * 200
DIM = 64

NUM_CORES = 2
NUM_SUBCORES = 16
NUM_WORKERS = NUM_CORES * NUM_SUBCORES

ROWS_PER_WORKER = ROWS // NUM_WORKERS
CHUNK = 512
CHUNKS_PER_WORKER = ROWS_PER_WORKER // CHUNK


def _embed_body(text_hbm, table_hbm, out_hbm, idx_v, rows_v, sem):
    wid = lax.axis_index("s") * NUM_CORES + lax.axis_index("c")
    base = wid * ROWS_PER_WORKER

    def chunk_step(i, carry):
        off = base + i * CHUNK
        pltpu.sync_copy(text_hbm.at[pl.ds(off, CHUNK)], idx_v)
        pltpu.async_copy(table_hbm.at[idx_v], rows_v, sem).wait()
        pltpu.sync_copy(rows_v, out_hbm.at[pl.ds(off, CHUNK)])
        return carry

    lax.fori_loop(0, CHUNKS_PER_WORKER, chunk_step, 0)


@jax.jit
def kernel(text, W):
    text_flat = text.reshape(-1).astype(jnp.int32)
    k = functools.partial(
        pl.kernel,
        mesh=plsc.VectorSubcoreMesh(core_axis_name="c", subcore_axis_name="s"),
        out_type=jax.ShapeDtypeStruct((ROWS, DIM), jnp.float32),
        scratch_types=[
            pltpu.VMEM((CHUNK,), jnp.int32),
            pltpu.VMEM((CHUNK, DIM), jnp.float32),
            pltpu.SemaphoreType.DMA,
        ],
        compiler_params=pltpu.CompilerParams(use_tc_tiling_on_sc=False),
    )(_embed_body)
    out = k(text_flat, W)
    return out.reshape(text.shape[0], text.shape[1], DIM)


# speedup vs baseline: 1.0369x; 1.0369x over previous
import functools

import jax
import jax.numpy as jnp
from jax import lax
from jax.experimental import pallas as pl
from jax.experimental.pallas import tpu as pltpu
from jax.experimental.pallas import tpu_sc as plsc

ROWS = 4096 * 200
DIM = 64

NUM_CORES = 2
NUM_SUBCORES = 16
NUM_WORKERS = NUM_CORES * NUM_SUBCORES

ROWS_PER_WORKER = ROWS // NUM_WORKERS
CHUNK = 320
NBUF = 4
CHUNKS_PER_WORKER = ROWS_PER_WORKER // CHUNK
OUTER = CHUNKS_PER_WORKER // NBUF


def _embed_body(text_hbm, table_hbm, out_hbm, idx_v, rows, gsems, wsems):
    wid = lax.axis_index("s") * NUM_CORES + lax.axis_index("c")
    base = wid * ROWS_PER_WORKER

    pltpu.sync_copy(text_hbm.at[pl.ds(base, ROWS_PER_WORKER)], idx_v)

    def start_gather(chunk_i, b):
        pltpu.async_copy(
            table_hbm.at[idx_v.at[pl.ds(chunk_i * CHUNK, CHUNK)]],
            rows[b],
            gsems[b],
        )

    def start_writeback(chunk_i, b):
        pltpu.async_copy(
            rows[b],
            out_hbm.at[pl.ds(base + chunk_i * CHUNK, CHUNK)],
            wsems[b],
        )

    def wait_gather(b):
        pltpu.make_async_copy(table_hbm.at[idx_v.at[pl.ds(0, CHUNK)]],
                              rows[b], gsems[b]).wait()

    def wait_writeback(b):
        pltpu.make_async_copy(rows[b], out_hbm.at[pl.ds(base, CHUNK)],
                              wsems[b]).wait()

    for b in range(NBUF):
        start_gather(b, b)

    def outer_step(g, carry):
        for b in range(NBUF):
            wait_gather(b)
            start_writeback(g * NBUF + b, b)
        for b in range(NBUF):
            wait_writeback(b)
            start_gather((g + 1) * NBUF + b, b)
        return carry

    lax.fori_loop(0, OUTER - 1, outer_step, 0)

    for b in range(NBUF):
        wait_gather(b)
        start_writeback((OUTER - 1) * NBUF + b, b)
    for b in range(NBUF):
        wait_writeback(b)


@jax.jit
def kernel(text, W):
    text_flat = text.reshape(-1).astype(jnp.int32)
    k = functools.partial(
        pl.kernel,
        mesh=plsc.VectorSubcoreMesh(core_axis_name="c", subcore_axis_name="s"),
        out_type=jax.ShapeDtypeStruct((ROWS, DIM), jnp.float32),
        scratch_types=[
            pltpu.VMEM((ROWS_PER_WORKER,), jnp.int32),
            [pltpu.VMEM((CHUNK, DIM), jnp.float32) for _ in range(NBUF)],
            [pltpu.SemaphoreType.DMA for _ in range(NBUF)],
            [pltpu.SemaphoreType.DMA for _ in range(NBUF)],
        ],
        compiler_params=pltpu.CompilerParams(use_tc_tiling_on_sc=False),
    )(_embed_body)
    out = k(text_flat, W)
    return out.reshape(text.shape[0], text.shape[1], DIM)
